# Initial kernel scaffold; baseline (speedup 1.0000x reference)
import jax
import jax.numpy as jnp
from jax.experimental import pallas as pl


def kernel(atom_feats, bond_feats, global_feats, atom_hop_distance, bond_hop_distance, atom_segment_ids, bond_segment_ids):
    raise NotImplementedError("write your pallas kernel here")


# speedup vs baseline: 4.9132x; 4.9132x over previous
import functools
import math

import jax
import jax.numpy as jnp
import numpy as np
from jax import lax
from jax.experimental import pallas as pl
from jax.experimental.pallas import tpu as pltpu
from jax.experimental.pallas import tpu_sc as plsc

MAXH = 3
G = 512
NCLS = 4
DUMP = NCLS * G
ACC_ROWS = 129 * 16
NC, NS = 2, 16
NW = NC * NS
CH = 128

_PI2 = np.float32(math.pi / 2.0)
W_HOP = tuple(
    float(np.float32(math.cos(np.float32(_PI2 * np.float32(h) / np.float32(3.0)))))
    for h in range(NCLS)
)


def _phase(w, feats, hop, seg, buf, segb, hopb, idxb, acc, n, S):
    lo = (((w * n) // NW) // 8) * 8
    hi = ((((w + 1) * n) // NW) // 8) * 8
    nfull = (hi - lo) // S
    max_rng = n // NW + 8
    nst = -(-max_rng // S)

    def stage(t, carry):
        full = t < nfull
        base = pl.multiple_of(jnp.where(full, lo + t * S, hi - S), 8)
        done = jnp.where(full, base, lo + nfull * S)
        pltpu.sync_copy(feats.at[pl.ds(base, S)], buf)
        pltpu.sync_copy(seg.at[pl.ds(base, S)], segb.at[pl.ds(0, S)])
        pltpu.sync_copy(hop.at[pl.ds(base, S)], hopb.at[pl.ds(0, S)])

        def idxk(k, c2):
            sg = segb[pl.ds(k * 16, 16)]
            hp = hopb[pl.ds(k * 16, 16)]
            hc = jnp.where((hp >= 1) & (hp <= MAXH), hp, 0)
            row = hc * G + sg
            gidx = base + k * 16 + lax.iota(jnp.int32, 16)
            row = jnp.where(gidx < done, DUMP, row)
            idxb[k // 8, pl.ds((k % 8) * 16, 16)] = row
            return c2

        lax.fori_loop(0, S // 16, idxk, 0)
        for j in range(S // CH):
            pltpu.sync_copy(buf.at[pl.ds(j * CH, CH)], acc.at[idxb.at[j]], add=True)
        return carry

    lax.fori_loop(0, nst, stage, 0)


@functools.lru_cache(maxsize=None)
def _make_sc(natom, nbond, da, db):
    sa = 512
    sb = 2048
    assert natom % 8 == 0 and nbond % 8 == 0
    assert natom // NW - 8 >= sa and nbond // NW - 8 >= sb

    mesh = plsc.VectorSubcoreMesh(core_axis_name="c", subcore_axis_name="s")

    @functools.partial(
        pl.kernel,
        out_type=(
            jax.ShapeDtypeStruct((NC, NCLS * G, da), jnp.float32),
            jax.ShapeDtypeStruct((NC, NCLS * G, db), jnp.float32),
        ),
        mesh=mesh,
        compiler_params=pltpu.CompilerParams(use_tc_tiling_on_sc=False),
        scratch_types=[
            pltpu.VMEM((sa, da), jnp.float32),
            pltpu.VMEM((sb, db), jnp.float32),
            pltpu.VMEM((sb,), jnp.int32),
            pltpu.VMEM((sb,), jnp.int32),
            pltpu.VMEM((sb // CH, CH), jnp.int32),
            pltpu.VMEM_SHARED((ACC_ROWS, da), jnp.float32),
            pltpu.VMEM_SHARED((ACC_ROWS, db), jnp.float32),
        ],
    )
    def sc(af, ah, asg, bf, bh, bsg, za, zb, pa, pb,
           abuf, bbuf, segb, hopb, idxb, acc_a, acc_b):
        c = lax.axis_index("c")
        s = lax.axis_index("s")
        w = c * NS + s
        pltpu.sync_copy(za, acc_a.at[pl.ds(s * 129, 129)])
        pltpu.sync_copy(zb, acc_b.at[pl.ds(s * 129, 129)])
        plsc.subcore_barrier()
        _phase(w, af, ah, asg, abuf, segb, hopb, idxb, acc_a, natom, sa)
        _phase(w, bf, bh, bsg, bbuf, segb, hopb, idxb, acc_b, nbond, sb)
        plsc.subcore_barrier()
        pltpu.sync_copy(acc_a.at[pl.ds(s * 128, 128)], abuf.at[pl.ds(0, 128)])
        pltpu.sync_copy(abuf.at[pl.ds(0, 128)], pa.at[c, pl.ds(s * 128, 128)])
        pltpu.sync_copy(acc_b.at[pl.ds(s * 128, 128)], bbuf.at[pl.ds(0, 128)])
        pltpu.sync_copy(bbuf.at[pl.ds(0, 128)], pb.at[c, pl.ds(s * 128, 128)])

    return sc


def _combine_body(pa_ref, pb_ref, oa_ref, ob_ref):
    oa = jnp.zeros((G, pa_ref.shape[-1]), jnp.float32)
    ob = jnp.zeros((G, pb_ref.shape[-1]), jnp.float32)
    for c in range(NC):
        for h in range(NCLS):
            oa = oa + W_HOP[h] * pa_ref[c, h * G:(h + 1) * G, :]
            ob = ob + W_HOP[h] * pb_ref[c, h * G:(h + 1) * G, :]
    oa_ref[...] = oa
    ob_ref[...] = ob


def kernel(atom_feats, bond_feats, global_feats, atom_hop_distance,
           bond_hop_distance, atom_segment_ids, bond_segment_ids):
    natom, da = atom_feats.shape
    nbond, db = bond_feats.shape
    za = jnp.zeros((129, da), jnp.float32)
    zb = jnp.zeros((129, db), jnp.float32)
    sc = _make_sc(natom, nbond, da, db)
    pa, pb = sc(atom_feats, atom_hop_distance.astype(jnp.int32),
                atom_segment_ids.astype(jnp.int32), bond_feats,
                bond_hop_distance.astype(jnp.int32),
                bond_segment_ids.astype(jnp.int32), za, zb)
    oa, ob = pl.pallas_call(
        _combine_body,
        out_shape=[
            jax.ShapeDtypeStruct((G, da), jnp.float32),
            jax.ShapeDtypeStruct((G, db), jnp.float32),
        ],
    )(pa, pb)
    return jnp.concatenate([oa, ob, global_feats], axis=-1)
